# baseline (speedup 1.0000x reference)
import functools

import jax
import jax.numpy as jnp
from jax import lax
from jax.experimental import pallas as pl
from jax.experimental.pallas import tpu as pltpu
from jax.experimental.pallas import tpu_sc as plsc

_IDX_CHUNK = 128


def _rowsum_body(tbl_ref, ones_ref, out_ref):
    out_ref[...] = jnp.dot(
        tbl_ref[...], ones_ref[...], preferred_element_type=jnp.float32
    )


def kernel(input, embeddings):
    V, D = embeddings.shape
    B = input.size

    vblk = 20000
    ones = jnp.ones((D, 1), jnp.float32)
    rowsums = pl.pallas_call(
        _rowsum_body,
        grid=(V // vblk,),
        in_specs=[
            pl.BlockSpec((vblk, D), lambda i: (i, 0)),
            pl.BlockSpec((D, 1), lambda i: (0, 0)),
        ],
        out_specs=pl.BlockSpec((vblk, 1), lambda i: (i, 0)),
        out_shape=jax.ShapeDtypeStruct((V, 1), jnp.float32),
    )(embeddings, ones)
    rowsums = rowsums.reshape(V)

    info = plsc.get_sparse_core_info()
    NC, NS, L = info.num_cores, info.num_subcores, info.num_lanes
    NW = NC * NS
    bpw = B // NW
    nchunks = bpw // _IDX_CHUNK
    idx = input.reshape(NW, nchunks, _IDX_CHUNK)

    @functools.partial(
        pl.kernel,
        mesh=plsc.VectorSubcoreMesh(core_axis_name="c", subcore_axis_name="s"),
        compiler_params=pltpu.CompilerParams(needs_layout_passes=False),
        out_type=jax.ShapeDtypeStruct((NW, L), jnp.float32),
        scratch_types=[
            pltpu.VMEM((nchunks, _IDX_CHUNK), jnp.int32),
            pltpu.VMEM((bpw,), jnp.float32),
            pltpu.VMEM((L,), jnp.float32),
            pltpu.SemaphoreType.DMA,
        ],
    )
    def _gather_sum(idx_hbm, rs_hbm, out_hbm, idx_v, vals_v, acc_v, sem):
        wid = lax.axis_index("s") * NC + lax.axis_index("c")
        pltpu.sync_copy(idx_hbm.at[wid], idx_v)

        def fire(j, carry):
            pltpu.make_async_copy(
                rs_hbm.at[idx_v.at[j]],
                vals_v.at[pl.ds(j * _IDX_CHUNK, _IDX_CHUNK)],
                sem,
            ).start()
            return carry

        lax.fori_loop(0, nchunks, fire, 0)

        def drain(j, carry):
            pltpu.make_async_copy(
                rs_hbm.at[idx_v.at[j]],
                vals_v.at[pl.ds(j * _IDX_CHUNK, _IDX_CHUNK)],
                sem,
            ).wait()
            return carry

        lax.fori_loop(0, nchunks, drain, 0)

        acc_v[...] = jnp.zeros((L,), jnp.float32)

        def body(j, carry):
            acc_v[...] = acc_v[...] + vals_v[pl.ds(j * L, L)]
            return carry

        lax.fori_loop(0, bpw // L, body, 0)
        pltpu.sync_copy(acc_v, out_hbm.at[wid])

    partials = _gather_sum(idx, rowsums)
    return jnp.sum(partials) / jnp.float32(B * D)


# speedup vs baseline: 1.0765x; 1.0765x over previous
import functools

import jax
import jax.numpy as jnp
from jax import lax
from jax.experimental import pallas as pl
from jax.experimental.pallas import tpu as pltpu
from jax.experimental.pallas import tpu_sc as plsc

_IDX_CHUNK = 128
_VBLK = 12800


def _rowsum_body(tbl_ref, out_ref):
    i = pl.program_id(0)
    s = jnp.sum(tbl_ref[...], axis=1)
    out_ref[pl.ds(i * _VBLK, _VBLK)] = s


def kernel(input, embeddings):
    V, D = embeddings.shape
    B = input.size

    nsteps = -(-V // _VBLK)
    Vpad = nsteps * _VBLK
    rowsums = pl.pallas_call(
        _rowsum_body,
        grid=(nsteps,),
        in_specs=[pl.BlockSpec((_VBLK, D), lambda i: (i, 0))],
        out_specs=pl.BlockSpec((Vpad,), lambda i: (0,)),
        out_shape=jax.ShapeDtypeStruct((Vpad,), jnp.float32),
    )(embeddings)

    info = plsc.get_sparse_core_info()
    NC, NS, L = info.num_cores, info.num_subcores, info.num_lanes
    NW = NC * NS
    bpw = B // NW
    nchunks = bpw // _IDX_CHUNK
    idx = input.reshape(NW, nchunks, _IDX_CHUNK)

    @functools.partial(
        pl.kernel,
        mesh=plsc.VectorSubcoreMesh(core_axis_name="c", subcore_axis_name="s"),
        compiler_params=pltpu.CompilerParams(needs_layout_passes=False),
        out_type=jax.ShapeDtypeStruct((NW, L), jnp.float32),
        scratch_types=[
            pltpu.VMEM((nchunks, _IDX_CHUNK), jnp.int32),
            pltpu.VMEM((bpw,), jnp.float32),
            pltpu.VMEM((L,), jnp.float32),
            pltpu.SemaphoreType.DMA,
        ],
    )
    def _gather_sum(idx_hbm, rs_hbm, out_hbm, idx_v, vals_v, acc_v, sem):
        wid = lax.axis_index("s") * NC + lax.axis_index("c")
        pltpu.sync_copy(idx_hbm.at[wid], idx_v)

        def fire(j, carry):
            pltpu.make_async_copy(
                rs_hbm.at[idx_v.at[j]],
                vals_v.at[pl.ds(j * _IDX_CHUNK, _IDX_CHUNK)],
                sem,
            ).start()
            return carry

        lax.fori_loop(0, nchunks, fire, 0)

        def drain(j, carry):
            pltpu.make_async_copy(
                rs_hbm.at[idx_v.at[j]],
                vals_v.at[pl.ds(j * _IDX_CHUNK, _IDX_CHUNK)],
                sem,
            ).wait()
            return carry

        lax.fori_loop(0, nchunks, drain, 0)

        acc_v[...] = jnp.zeros((L,), jnp.float32)

        def body(j, carry):
            acc_v[...] = acc_v[...] + vals_v[pl.ds(j * L, L)]
            return carry

        lax.fori_loop(0, bpw // L, body, 0)
        pltpu.sync_copy(acc_v, out_hbm.at[wid])

    partials = _gather_sum(idx, rowsums)
    return jnp.sum(partials) / jnp.float32(B * D)


# speedup vs baseline: 1.5309x; 1.4220x over previous
import functools

import jax
import jax.numpy as jnp
from jax import lax
from jax.experimental import pallas as pl
from jax.experimental.pallas import tpu as pltpu
from jax.experimental.pallas import tpu_sc as plsc

_IDX_CHUNK = 128
_VBLK = 16384


def _rowsum_body(tbl_ref, out_ref):
    s = jnp.sum(tbl_ref[...], axis=1)
    out_ref[...] = s.reshape(_VBLK // 128, 128)


def kernel(input, embeddings):
    V, D = embeddings.shape
    B = input.size

    nsteps = -(-V // _VBLK)
    Vpad = nsteps * _VBLK
    rowsums = pl.pallas_call(
        _rowsum_body,
        grid=(nsteps,),
        in_specs=[pl.BlockSpec((_VBLK, D), lambda i: (i, 0))],
        out_specs=pl.BlockSpec((_VBLK // 128, 128), lambda i: (i, 0)),
        out_shape=jax.ShapeDtypeStruct((Vpad // 128, 128), jnp.float32),
    )(embeddings)
    rowsums = rowsums.reshape(Vpad)

    info = plsc.get_sparse_core_info()
    NC, NS, L = info.num_cores, info.num_subcores, info.num_lanes
    NW = NC * NS
    bpw = B // NW
    nchunks = bpw // _IDX_CHUNK
    idx = input.reshape(B)

    @functools.partial(
        pl.kernel,
        mesh=plsc.VectorSubcoreMesh(core_axis_name="c", subcore_axis_name="s"),
        compiler_params=pltpu.CompilerParams(needs_layout_passes=False),
        out_type=jax.ShapeDtypeStruct((NW, L), jnp.float32),
        scratch_types=[
            pltpu.VMEM((bpw,), jnp.int32),
            pltpu.VMEM((bpw,), jnp.float32),
            pltpu.VMEM((L,), jnp.float32),
            pltpu.SemaphoreType.DMA,
        ],
    )
    def _gather_sum(idx_hbm, rs_hbm, out_hbm, idx_v, vals_v, acc_v, sem):
        wid = lax.axis_index("s") * NC + lax.axis_index("c")
        pltpu.sync_copy(idx_hbm.at[pl.ds(wid * bpw, bpw)], idx_v)

        def fire(j, carry):
            pltpu.make_async_copy(
                rs_hbm.at[idx_v.at[pl.ds(j * _IDX_CHUNK, _IDX_CHUNK)]],
                vals_v.at[pl.ds(j * _IDX_CHUNK, _IDX_CHUNK)],
                sem,
            ).start()
            return carry

        lax.fori_loop(0, nchunks, fire, 0)

        def drain(j, carry):
            pltpu.make_async_copy(
                rs_hbm.at[idx_v.at[pl.ds(j * _IDX_CHUNK, _IDX_CHUNK)]],
                vals_v.at[pl.ds(j * _IDX_CHUNK, _IDX_CHUNK)],
                sem,
            ).wait()
            return carry

        lax.fori_loop(0, nchunks, drain, 0)

        def body(j, acc):
            base = j * 4 * L
            for k in range(4):
                acc = acc + vals_v[pl.ds(base + k * L, L)]
            return acc

        acc = lax.fori_loop(
            0, bpw // (4 * L), body, jnp.zeros((L,), jnp.float32)
        )
        acc_v[...] = acc
        pltpu.sync_copy(acc_v, out_hbm.at[wid])

    partials = _gather_sum(idx, rowsums)
    return jnp.sum(partials) / jnp.float32(B * D)
